# baseline (speedup 1.0000x reference)
import functools

import jax
import jax.numpy as jnp
from jax import lax
from jax.experimental import pallas as pl
from jax.experimental.pallas import tpu as pltpu
from jax.experimental.pallas import tpu_sc as plsc

BATCH = 4096
SEQ_LEN = 50
EMBED_DIM = 64
ROWS = BATCH * SEQ_LEN
NPAIR_ROWS = 500000
NC, NS = 2, 16
NW = NC * NS
ROWS_PER_W = ROWS // NW
CHUNK = 128
NCHUNK = ROWS_PER_W // CHUNK


@functools.cache
def _make_gather():
    mesh = plsc.VectorSubcoreMesh(core_axis_name="c", subcore_axis_name="s",
                                  num_cores=NC, num_subcores=NS)

    @functools.partial(
        pl.kernel,
        mesh=mesh,
        out_type=jax.ShapeDtypeStruct((ROWS, 2 * EMBED_DIM), jnp.float32),
        scratch_types=[
            pltpu.VMEM((NCHUNK, CHUNK), jnp.int32),
            pltpu.VMEM((CHUNK, 2 * EMBED_DIM), jnp.float32),
            pltpu.SemaphoreType.DMA,
        ],
    )
    def gather_k(ids_hbm, table_hbm, out_hbm, idx_v, rows_v, sem):
        wid = lax.axis_index("s") * NC + lax.axis_index("c")
        pltpu.sync_copy(ids_hbm.at[wid], idx_v)
        base = wid * ROWS_PER_W

        def body(j, _):
            pltpu.async_copy(table_hbm.at[idx_v.at[j]], rows_v, sem).wait()
            pltpu.sync_copy(rows_v, out_hbm.at[pl.ds(base + j * CHUNK, CHUNK)])
            return 0

        lax.fori_loop(0, NCHUNK, body, 0)

    return gather_k


_BLK = 2048


def _mm_body(ue_ref, pairs_ref, h_ref, w1_ref, w2_ref, b_ref, out_ref):
    left = pairs_ref[:, :EMBED_DIM]
    right = pairs_ref[:, EMBED_DIM:]
    h = h_ref[...].astype(jnp.float32)
    soc = left + (right - left) * h
    acc = jnp.dot(ue_ref[...], w1_ref[...], preferred_element_type=jnp.float32)
    acc += jnp.dot(soc, w2_ref[...], preferred_element_type=jnp.float32)
    out_ref[...] = jnp.maximum(acc + b_ref[...], 0.0)


def _fused_linear(ue, pairs, hmask, w1t, w2t, b2d):
    return pl.pallas_call(
        _mm_body,
        grid=(ROWS // _BLK,),
        in_specs=[
            pl.BlockSpec((_BLK, EMBED_DIM), lambda i: (i, 0)),
            pl.BlockSpec((_BLK, 2 * EMBED_DIM), lambda i: (i, 0)),
            pl.BlockSpec((_BLK, EMBED_DIM), lambda i: (i, 0)),
            pl.BlockSpec((EMBED_DIM, EMBED_DIM), lambda i: (0, 0)),
            pl.BlockSpec((EMBED_DIM, EMBED_DIM), lambda i: (0, 0)),
            pl.BlockSpec((1, EMBED_DIM), lambda i: (0, 0)),
        ],
        out_specs=pl.BlockSpec((_BLK, EMBED_DIM), lambda i: (i, 0)),
        out_shape=jax.ShapeDtypeStruct((ROWS, EMBED_DIM), jnp.float32),
        compiler_params=pltpu.CompilerParams(
            dimension_semantics=("arbitrary",)),
    )(ue, pairs, hmask, w1t, w2t, b2d)


def kernel(user_embeds, user_ids, emb_table, W, b):
    idflat = user_ids.astype(jnp.int32).reshape(ROWS)
    pid = (idflat >> 1).reshape(NW, NCHUNK, CHUNK)
    hmask = jnp.broadcast_to((idflat & 1).astype(jnp.int8)[:, None],
                             (ROWS, EMBED_DIM))
    pairs_view = emb_table.reshape(NPAIR_ROWS, 2 * EMBED_DIM)
    social_pairs = _make_gather()(pid, pairs_view)
    ue = user_embeds.reshape(ROWS, EMBED_DIM)
    wt = W.T
    out = _fused_linear(ue, social_pairs, hmask, wt[:EMBED_DIM],
                        wt[EMBED_DIM:], b.reshape(1, EMBED_DIM))
    return out.reshape(BATCH, SEQ_LEN, EMBED_DIM)


# speedup vs baseline: 1.6571x; 1.6571x over previous
import functools

import jax
import jax.numpy as jnp
from jax import lax
from jax.experimental import pallas as pl
from jax.experimental.pallas import tpu as pltpu
from jax.experimental.pallas import tpu_sc as plsc

BATCH = 4096
SEQ_LEN = 50
EMBED_DIM = 64
ROWS = BATCH * SEQ_LEN
NTILE = 125000
NC, NS = 2, 16
NW = NC * NS
ROWS_PER_W = ROWS // NW
CHUNK = 128
NCHUNK = ROWS_PER_W // CHUNK


@functools.cache
def _make_gather():
    mesh = plsc.VectorSubcoreMesh(core_axis_name="c", subcore_axis_name="s",
                                  num_cores=NC, num_subcores=NS)

    @functools.partial(
        pl.kernel,
        mesh=mesh,
        out_type=jax.ShapeDtypeStruct((ROWS, EMBED_DIM), jnp.float32),
        scratch_types=[
            pltpu.VMEM((CHUNK,), jnp.int32),
            pltpu.VMEM((CHUNK, EMBED_DIM), jnp.float32),
            pltpu.SemaphoreType.DMA,
        ],
    )
    def gather_k(ids_hbm, table_hbm, out_hbm, idx_v, rows_v, sem):
        wid = lax.axis_index("s") * NC + lax.axis_index("c")
        base = wid * ROWS_PER_W

        def chunk_body(j, _):
            pltpu.sync_copy(ids_hbm.at[wid, j], idx_v)

            def grp_body(g, _):
                vids = idx_v[pl.ds(g * 16, 16)]
                for k in range(16):
                    rid = vids[k]
                    pltpu.async_copy(table_hbm.at[rid >> 3, rid & 7],
                                     rows_v.at[g * 16 + k], sem)
                return 0

            lax.fori_loop(0, CHUNK // 16, grp_body, 0)

            def drain_body(r, _):
                pltpu.make_async_copy(table_hbm.at[0, 0], rows_v.at[r],
                                      sem).wait()
                return 0

            lax.fori_loop(0, CHUNK, drain_body, 0)
            out_slice = out_hbm.at[pl.ds(base + j * CHUNK, CHUNK)]
            pltpu.sync_copy(rows_v, out_slice)
            return 0

        lax.fori_loop(0, NCHUNK, chunk_body, 0)

    return gather_k


_BLK = 2048


def _mm_body(ue_ref, soc_ref, w1_ref, w2_ref, b_ref, out_ref):
    acc = jnp.dot(ue_ref[...], w1_ref[...], preferred_element_type=jnp.float32)
    acc += jnp.dot(soc_ref[...], w2_ref[...], preferred_element_type=jnp.float32)
    out_ref[...] = jnp.maximum(acc + b_ref[...], 0.0)


def _fused_linear(ue, soc, w1t, w2t, b2d):
    return pl.pallas_call(
        _mm_body,
        grid=(ROWS // _BLK,),
        in_specs=[
            pl.BlockSpec((_BLK, EMBED_DIM), lambda i: (i, 0)),
            pl.BlockSpec((_BLK, EMBED_DIM), lambda i: (i, 0)),
            pl.BlockSpec((EMBED_DIM, EMBED_DIM), lambda i: (0, 0)),
            pl.BlockSpec((EMBED_DIM, EMBED_DIM), lambda i: (0, 0)),
            pl.BlockSpec((1, EMBED_DIM), lambda i: (0, 0)),
        ],
        out_specs=pl.BlockSpec((_BLK, EMBED_DIM), lambda i: (i, 0)),
        out_shape=jax.ShapeDtypeStruct((ROWS, EMBED_DIM), jnp.float32),
        compiler_params=pltpu.CompilerParams(
            dimension_semantics=("arbitrary",)),
    )(ue, soc, w1t, w2t, b2d)


def kernel(user_embeds, user_ids, emb_table, W, b):
    ids = user_ids.astype(jnp.int32).reshape(NW, NCHUNK, CHUNK)
    table3 = emb_table.reshape(NTILE, 8, EMBED_DIM)
    social = _make_gather()(ids, table3)
    ue = user_embeds.reshape(ROWS, EMBED_DIM)
    wt = W.T
    out = _fused_linear(ue, social, wt[:EMBED_DIM], wt[EMBED_DIM:],
                        b.reshape(1, EMBED_DIM))
    return out.reshape(BATCH, SEQ_LEN, EMBED_DIM)


# speedup vs baseline: 1.6590x; 1.0012x over previous
import functools

import jax
import jax.numpy as jnp
from jax import lax
from jax.experimental import pallas as pl
from jax.experimental.pallas import tpu as pltpu
from jax.experimental.pallas import tpu_sc as plsc

BATCH = 4096
SEQ_LEN = 50
EMBED_DIM = 64
ROWS = BATCH * SEQ_LEN
NTILE = 125000
NC, NS = 2, 16
NW = NC * NS
ROWS_PER_W = ROWS // NW
CHUNK = 128
NCHUNK = ROWS_PER_W // CHUNK


@functools.cache
def _make_gather():
    mesh = plsc.VectorSubcoreMesh(core_axis_name="c", subcore_axis_name="s",
                                  num_cores=NC, num_subcores=NS)

    @functools.partial(
        pl.kernel,
        mesh=mesh,
        out_type=jax.ShapeDtypeStruct((ROWS, EMBED_DIM), jnp.float32),
        scratch_types=[
            pltpu.VMEM((CHUNK,), jnp.int32),
            pltpu.VMEM((CHUNK, EMBED_DIM), jnp.float32),
            pltpu.SemaphoreType.DMA,
        ],
        compiler_params=pltpu.CompilerParams(use_tc_tiling_on_sc=True),
    )
    def gather_k(ids_hbm, table_hbm, out_hbm, idx_v, rows_v, sem):
        wid = lax.axis_index("s") * NC + lax.axis_index("c")
        base = wid * ROWS_PER_W

        def chunk_body(j, _):
            pltpu.sync_copy(ids_hbm.at[wid, j], idx_v)

            def grp_body(g, _):
                vids = idx_v[pl.ds(g * 16, 16)]
                for k in range(16):
                    rid = vids[k]
                    pltpu.async_copy(table_hbm.at[rid >> 3, rid & 7],
                                     rows_v.at[g * 16 + k], sem)
                return 0

            lax.fori_loop(0, CHUNK // 16, grp_body, 0)

            def drain_body(r, _):
                pltpu.make_async_copy(table_hbm.at[0, 0], rows_v.at[r],
                                      sem).wait()
                return 0

            lax.fori_loop(0, CHUNK, drain_body, 0)
            out_slice = out_hbm.at[pl.ds(base + j * CHUNK, CHUNK)]
            pltpu.sync_copy(rows_v, out_slice)
            return 0

        lax.fori_loop(0, NCHUNK, chunk_body, 0)

    return gather_k


_BLK = 2048


def _mm_body(ue_ref, soc_ref, w1_ref, w2_ref, b_ref, out_ref):
    acc = jnp.dot(ue_ref[...], w1_ref[...], preferred_element_type=jnp.float32)
    acc += jnp.dot(soc_ref[...], w2_ref[...], preferred_element_type=jnp.float32)
    out_ref[...] = jnp.maximum(acc + b_ref[...], 0.0)


def _fused_linear(ue, soc, w1t, w2t, b2d):
    return pl.pallas_call(
        _mm_body,
        grid=(ROWS // _BLK,),
        in_specs=[
            pl.BlockSpec((_BLK, EMBED_DIM), lambda i: (i, 0)),
            pl.BlockSpec((_BLK, EMBED_DIM), lambda i: (i, 0)),
            pl.BlockSpec((EMBED_DIM, EMBED_DIM), lambda i: (0, 0)),
            pl.BlockSpec((EMBED_DIM, EMBED_DIM), lambda i: (0, 0)),
            pl.BlockSpec((1, EMBED_DIM), lambda i: (0, 0)),
        ],
        out_specs=pl.BlockSpec((_BLK, EMBED_DIM), lambda i: (i, 0)),
        out_shape=jax.ShapeDtypeStruct((ROWS, EMBED_DIM), jnp.float32),
        compiler_params=pltpu.CompilerParams(
            dimension_semantics=("arbitrary",)),
    )(ue, soc, w1t, w2t, b2d)


def kernel(user_embeds, user_ids, emb_table, W, b):
    ids = user_ids.astype(jnp.int32).reshape(NW, NCHUNK, CHUNK)
    table3 = emb_table.reshape(NTILE, 8, EMBED_DIM)
    social = _make_gather()(ids, table3)
    ue = user_embeds.reshape(ROWS, EMBED_DIM)
    wt = W.T
    out = _fused_linear(ue, social, wt[:EMBED_DIM], wt[EMBED_DIM:],
                        b.reshape(1, EMBED_DIM))
    return out.reshape(BATCH, SEQ_LEN, EMBED_DIM)
